# Initial kernel scaffold; baseline (speedup 1.0000x reference)
import jax
import jax.numpy as jnp
from jax.experimental import pallas as pl


def kernel(x, embed):
    raise NotImplementedError("write your pallas kernel here")


# speedup vs baseline: 3.1135x; 3.1135x over previous
import functools

import jax
import jax.numpy as jnp
from jax import lax
from jax.experimental import pallas as pl
from jax.experimental.pallas import tpu as pltpu

_K = 512
_C = 32
_BLK = 1024


def _rowsum_ref_order(sq):
    p = ((sq[:, 0:8] + sq[:, 8:16]) + sq[:, 16:24]) + sq[:, 24:32]
    t = p[:, 0:4] + p[:, 4:8]
    u = t[:, 0:2] + t[:, 2:4]
    return u[:, 0:1] + u[:, 1:2]


def _vq_block(x_ref, emb_ref, ind_ref, q_ref):
    x = x_ref[...]
    emb = emb_ref[...]
    en = lax.dot_general(jnp.ones((1, _C), jnp.float32), emb * emb,
                         (((1,), (1,)), ((), ())),
                         precision=lax.Precision.HIGHEST,
                         preferred_element_type=jnp.float32)
    s = lax.dot_general(x, emb, (((1,), (1,)), ((), ())),
                        precision=lax.Precision.HIGHEST,
                        preferred_element_type=jnp.float32)
    approx = en - 2.0 * s
    iota = lax.broadcasted_iota(jnp.int32, approx.shape, 1)
    m1 = jnp.min(approx, axis=1, keepdims=True)
    i1 = jnp.min(jnp.where(approx == m1, iota, _K), axis=1, keepdims=True)
    masked = jnp.where(iota == i1, jnp.inf, approx)
    m2 = jnp.min(masked, axis=1, keepdims=True)
    i2 = jnp.min(jnp.where(masked == m2, iota, _K), axis=1, keepdims=True)
    e1 = lax.dot_general((iota == i1).astype(jnp.float32), emb,
                         (((1,), (0,)), ((), ())),
                         precision=lax.Precision.HIGHEST,
                         preferred_element_type=jnp.float32)
    e2 = lax.dot_general((iota == i2).astype(jnp.float32), emb,
                         (((1,), (0,)), ((), ())),
                         precision=lax.Precision.HIGHEST,
                         preferred_element_type=jnp.float32)
    d1 = _rowsum_ref_order((x - e1) ** 2)
    d2 = _rowsum_ref_order((x - e2) ** 2)
    take2 = (d2 < d1) | ((d2 == d1) & (i2 < i1))
    ind_ref[...] = jnp.where(take2, i2, i1)
    q_ref[...] = jnp.where(take2, e2, e1)


@jax.jit
def _vq(xf, embed):
    n = xf.shape[0]
    grid = (n // _BLK,)
    ind, q = pl.pallas_call(
        _vq_block,
        grid=grid,
        in_specs=[
            pl.BlockSpec((_BLK, _C), lambda i: (i, 0)),
            pl.BlockSpec((_K, _C), lambda i: (0, 0)),
        ],
        out_specs=[
            pl.BlockSpec((_BLK, 1), lambda i: (i, 0)),
            pl.BlockSpec((_BLK, _C), lambda i: (i, 0)),
        ],
        out_shape=[
            jax.ShapeDtypeStruct((n, 1), jnp.int32),
            jax.ShapeDtypeStruct((n, _C), jnp.float32),
        ],
    )(xf, embed)
    return ind, q


def kernel(x, embed):
    B, T, C = x.shape
    xf = x.reshape(B * T, C)
    ind, q = _vq(xf, embed)
    return q.reshape(B, T, C), ind.reshape(B, T)
